# baseline (speedup 1.0000x reference)
import functools

import jax
import jax.numpy as jnp
from jax import lax
from jax.experimental import pallas as pl
from jax.experimental.pallas import tpu as pltpu
from jax.experimental.pallas import tpu_sc as plsc

DIM = 64
LANES = 16
NW = 32
CHUNK = 512
IDXW = 128


def _sc_body(pos_u_hbm, posv_hbm, negv_hbm, uw_hbm, vw_hbm, out_hbm,
             uidx_v, u_v, pidx_v, nidx_v, rows_v, part_v, sem,
             *, pos_per_w, neg_per_w):
    wid = lax.axis_index("s") * 2 + lax.axis_index("c")

    pltpu.sync_copy(pos_u_hbm, uidx_v)
    pltpu.async_copy(uw_hbm.at[uidx_v], u_v, sem).wait()
    u0 = u_v[0, pl.ds(0, LANES)]
    u1 = u_v[0, pl.ds(LANES, LANES)]
    u2 = u_v[0, pl.ds(2 * LANES, LANES)]
    u3 = u_v[0, pl.ds(3 * LANES, LANES)]

    zero = jnp.zeros((LANES,), jnp.float32)
    lane = lax.iota(jnp.int32, LANES)

    def gather_chunk(idx2d, first_row):
        descs = []
        for j in range(CHUNK // IDXW):
            descs.append(pltpu.async_copy(
                vw_hbm.at[idx2d.at[first_row + j]],
                rows_v.at[pl.ds(j * IDXW, IDXW)], sem))
        for d in descs:
            d.wait()

    pltpu.sync_copy(posv_hbm.at[wid], pidx_v)
    pos_part = jnp.zeros((LANES,), jnp.float32)
    for c in range(pos_per_w // CHUNK):
        gather_chunk(pidx_v, c * (CHUNK // IDXW))

        def pos_iter(i, accs):
            a0, a1, a2, a3 = accs
            for j in range(8):
                r = i * 8 + j
                a0 = a0 + rows_v[r, pl.ds(0, LANES)]
                a1 = a1 + rows_v[r, pl.ds(LANES, LANES)]
                a2 = a2 + rows_v[r, pl.ds(2 * LANES, LANES)]
                a3 = a3 + rows_v[r, pl.ds(3 * LANES, LANES)]
            return (a0, a1, a2, a3)

        a0, a1, a2, a3 = lax.fori_loop(
            0, CHUNK // 8, pos_iter, (zero, zero, zero, zero))
        pos_part = pos_part + a0 * u0 + a1 * u1 + a2 * u2 + a3 * u3

    pltpu.sync_copy(negv_hbm.at[wid], nidx_v)
    negacc = jnp.zeros((LANES,), jnp.float32)
    for c in range(neg_per_w // CHUNK):
        gather_chunk(nidx_v, c * (CHUNK // IDXW))

        def neg_iter(g, acc):
            rowids = lane + g * LANES
            dots = zero
            uvecs = (u0, u1, u2, u3)
            for d in range(DIM):
                col = jnp.full((LANES,), d, jnp.int32)
                vals = plsc.load_gather(rows_v, [rowids, col])
                dots = dots + vals * uvecs[d // LANES][d % LANES]
            return acc + jnp.exp(dots)

        negacc = lax.fori_loop(0, CHUNK // LANES, neg_iter, negacc)

    part_v[0, :] = pos_part
    part_v[1, :] = negacc
    pltpu.sync_copy(part_v, out_hbm.at[wid])


@functools.partial(jax.jit, static_argnames=("pos_per_w", "neg_per_w"))
def _sc_call(pos_u, posv, negv, u_weight, v_weight, *, pos_per_w, neg_per_w):
    body = functools.partial(_sc_body, pos_per_w=pos_per_w, neg_per_w=neg_per_w)
    return pl.kernel(
        body,
        out_type=jax.ShapeDtypeStruct((NW, 2, LANES), jnp.float32),
        mesh=plsc.VectorSubcoreMesh(core_axis_name="c", subcore_axis_name="s",
                                    num_cores=2, num_subcores=16),
        compiler_params=pltpu.CompilerParams(needs_layout_passes=False,
                                             use_tc_tiling_on_sc=False),
        scratch_types=[
            pltpu.VMEM((1,), jnp.int32),
            pltpu.VMEM((1, DIM), jnp.float32),
            pltpu.VMEM((pos_per_w // IDXW, IDXW), jnp.int32),
            pltpu.VMEM((neg_per_w // IDXW, IDXW), jnp.int32),
            pltpu.VMEM((CHUNK, DIM), jnp.float32),
            pltpu.VMEM((2, LANES), jnp.float32),
            pltpu.SemaphoreType.DMA,
        ],
    )(pos_u, posv, negv, u_weight, v_weight)


def kernel(pos_u, pos_v, neg_v, batch_size, u_weight, v_weight):
    B = pos_v.shape[0]
    N = neg_v.shape[0]
    pos_per_w = B // NW
    neg_per_w = N // NW
    posv = pos_v.reshape(NW, pos_per_w // IDXW, IDXW)
    negv = neg_v.reshape(NW, neg_per_w // IDXW, IDXW)
    parts = _sc_call(pos_u.astype(jnp.int32), posv, negv, u_weight, v_weight,
                     pos_per_w=pos_per_w, neg_per_w=neg_per_w)
    pos_sum = jnp.sum(parts[:, 0, :])
    s = jnp.sum(parts[:, 1, :])
    return -pos_sum + jnp.log(1.0 + s)


# speedup vs baseline: 1.0659x; 1.0659x over previous
import functools

import jax
import jax.numpy as jnp
from jax import lax
from jax.experimental import pallas as pl
from jax.experimental.pallas import tpu as pltpu
from jax.experimental.pallas import tpu_sc as plsc

DIM = 64
LANES = 16
NW = 32
CHUNK = 512
IDXW = 128


def _sc_body(pos_u_hbm, posv_hbm, negv_hbm, uw_hbm, vw_hbm, out_hbm,
             uidx_v, u_v, pidx_v, nidx_v, rows_v, part_v, sem,
             *, pos_per_w, neg_per_w):
    wid = lax.axis_index("s") * 2 + lax.axis_index("c")

    pltpu.sync_copy(pos_u_hbm, uidx_v)
    pltpu.async_copy(uw_hbm.at[uidx_v], u_v, sem).wait()
    u0 = u_v[0, pl.ds(0, LANES)]
    u1 = u_v[0, pl.ds(LANES, LANES)]
    u2 = u_v[0, pl.ds(2 * LANES, LANES)]
    u3 = u_v[0, pl.ds(3 * LANES, LANES)]

    zero = jnp.zeros((LANES,), jnp.float32)
    lane15 = lax.iota(jnp.int32, LANES) == 15

    def gather_chunk(idx2d, first_row):
        descs = []
        for j in range(CHUNK // IDXW):
            descs.append(pltpu.async_copy(
                vw_hbm.at[idx2d.at[first_row + j]],
                rows_v.at[pl.ds(j * IDXW, IDXW)], sem))
        for d in descs:
            d.wait()

    pltpu.sync_copy(posv_hbm.at[wid], pidx_v)
    pos_part = jnp.zeros((LANES,), jnp.float32)
    for c in range(pos_per_w // CHUNK):
        gather_chunk(pidx_v, c * (CHUNK // IDXW))

        def pos_iter(i, accs):
            a0, a1, a2, a3 = accs
            for j in range(8):
                r = i * 8 + j
                a0 = a0 + rows_v[r, pl.ds(0, LANES)]
                a1 = a1 + rows_v[r, pl.ds(LANES, LANES)]
                a2 = a2 + rows_v[r, pl.ds(2 * LANES, LANES)]
                a3 = a3 + rows_v[r, pl.ds(3 * LANES, LANES)]
            return (a0, a1, a2, a3)

        a0, a1, a2, a3 = lax.fori_loop(
            0, CHUNK // 8, pos_iter, (zero, zero, zero, zero))
        pos_part = pos_part + a0 * u0 + a1 * u1 + a2 * u2 + a3 * u3

    pltpu.sync_copy(negv_hbm.at[wid], nidx_v)
    negacc = jnp.zeros((LANES,), jnp.float32)
    for c in range(neg_per_w // CHUNK):
        gather_chunk(nidx_v, c * (CHUNK // IDXW))

        def neg_iter(i, acc):
            for j in range(8):
                r = i * 8 + j
                q = (rows_v[r, pl.ds(0, LANES)] * u0
                     + rows_v[r, pl.ds(LANES, LANES)] * u1
                     + rows_v[r, pl.ds(2 * LANES, LANES)] * u2
                     + rows_v[r, pl.ds(3 * LANES, LANES)] * u3)
                acc = acc + jnp.where(lane15, jnp.exp(plsc.cumsum(q)), zero)
            return acc

        negacc = lax.fori_loop(0, CHUNK // 8, neg_iter, negacc)

    part_v[0, :] = pos_part
    part_v[1, :] = negacc
    pltpu.sync_copy(part_v, out_hbm.at[wid])


@functools.partial(jax.jit, static_argnames=("pos_per_w", "neg_per_w"))
def _sc_call(pos_u, posv, negv, u_weight, v_weight, *, pos_per_w, neg_per_w):
    body = functools.partial(_sc_body, pos_per_w=pos_per_w, neg_per_w=neg_per_w)
    return pl.kernel(
        body,
        out_type=jax.ShapeDtypeStruct((NW, 2, LANES), jnp.float32),
        mesh=plsc.VectorSubcoreMesh(core_axis_name="c", subcore_axis_name="s",
                                    num_cores=2, num_subcores=16),
        compiler_params=pltpu.CompilerParams(needs_layout_passes=False,
                                             use_tc_tiling_on_sc=False),
        scratch_types=[
            pltpu.VMEM((1,), jnp.int32),
            pltpu.VMEM((1, DIM), jnp.float32),
            pltpu.VMEM((pos_per_w // IDXW, IDXW), jnp.int32),
            pltpu.VMEM((neg_per_w // IDXW, IDXW), jnp.int32),
            pltpu.VMEM((CHUNK, DIM), jnp.float32),
            pltpu.VMEM((2, LANES), jnp.float32),
            pltpu.SemaphoreType.DMA,
        ],
    )(pos_u, posv, negv, u_weight, v_weight)


def kernel(pos_u, pos_v, neg_v, batch_size, u_weight, v_weight):
    B = pos_v.shape[0]
    N = neg_v.shape[0]
    pos_per_w = B // NW
    neg_per_w = N // NW
    posv = pos_v.reshape(NW, pos_per_w // IDXW, IDXW)
    negv = neg_v.reshape(NW, neg_per_w // IDXW, IDXW)
    parts = _sc_call(pos_u.astype(jnp.int32), posv, negv, u_weight, v_weight,
                     pos_per_w=pos_per_w, neg_per_w=neg_per_w)
    pos_sum = jnp.sum(parts[:, 0, :])
    s = jnp.sum(parts[:, 1, :])
    return -pos_sum + jnp.log(1.0 + s)


# speedup vs baseline: 5.5812x; 5.2360x over previous
import functools

import jax
import jax.numpy as jnp
from jax import lax
from jax.experimental import pallas as pl
from jax.experimental.pallas import tpu as pltpu
from jax.experimental.pallas import tpu_sc as plsc

DIM = 64
LANES = 16
NW = 32
BLK = 4096
IDXW = 128


def _matvec_body(pu_ref, ut_blk, vt_blk, out_blk):
    j = pu_ref[0] % 128
    col = lax.broadcasted_iota(jnp.int32, (DIM, 128), 1)
    ucol = jnp.sum(jnp.where(col == j, ut_blk[...], 0.0), axis=1,
                   keepdims=True)
    out_blk[...] = jnp.sum(vt_blk[...] * ucol, axis=0)


def _tc_scores(pos_u, ut, vt, vocab):
    grid = (pl.cdiv(vocab, BLK),)
    return pl.pallas_call(
        _matvec_body,
        grid_spec=pltpu.PrefetchScalarGridSpec(
            num_scalar_prefetch=1,
            grid=grid,
            in_specs=[
                pl.BlockSpec((DIM, 128), lambda i, pu: (0, pu[0] // 128)),
                pl.BlockSpec((DIM, BLK), lambda i, pu: (0, i)),
            ],
            out_specs=pl.BlockSpec((BLK,), lambda i, pu: (i,)),
        ),
        out_shape=jax.ShapeDtypeStruct((vocab,), jnp.float32),
    )(pos_u, ut, vt)


def _sc_body(scores_hbm, posv_hbm, negv_hbm, out_hbm,
             pidx_v, nidx_v, pvals_v, nvals_v, part_v, sem,
             *, pos_per_w, neg_per_w):
    wid = lax.axis_index("s") * 2 + lax.axis_index("c")

    pltpu.sync_copy(posv_hbm.at[pl.ds(wid * pos_per_w, pos_per_w)], pidx_v)
    pltpu.sync_copy(negv_hbm.at[pl.ds(wid * neg_per_w, neg_per_w)], nidx_v)

    descs = []
    for j in range(pos_per_w // IDXW):
        descs.append(pltpu.async_copy(
            scores_hbm.at[pidx_v.at[pl.ds(j * IDXW, IDXW)]],
            pvals_v.at[pl.ds(j * IDXW, IDXW)], sem))
    for j in range(neg_per_w // IDXW):
        descs.append(pltpu.async_copy(
            scores_hbm.at[nidx_v.at[pl.ds(j * IDXW, IDXW)]],
            nvals_v.at[pl.ds(j * IDXW, IDXW)], sem))
    for d in descs:
        d.wait()

    zero = jnp.zeros((LANES,), jnp.float32)

    def pos_iter(i, acc):
        return acc + pvals_v[pl.ds(i * LANES, LANES)]

    pos_acc = lax.fori_loop(0, pos_per_w // LANES, pos_iter, zero)

    def neg_iter(i, acc):
        return acc + jnp.exp(nvals_v[pl.ds(i * LANES, LANES)])

    neg_acc = lax.fori_loop(0, neg_per_w // LANES, neg_iter, zero)

    part_v[0, :] = pos_acc
    part_v[1, :] = neg_acc
    pltpu.sync_copy(part_v, out_hbm.at[wid])


@functools.partial(jax.jit, static_argnames=("pos_per_w", "neg_per_w"))
def _sc_reduce(scores, pos_v, neg_v, *, pos_per_w, neg_per_w):
    body = functools.partial(_sc_body, pos_per_w=pos_per_w,
                             neg_per_w=neg_per_w)
    return pl.kernel(
        body,
        out_type=jax.ShapeDtypeStruct((NW, 2, LANES), jnp.float32),
        mesh=plsc.VectorSubcoreMesh(core_axis_name="c", subcore_axis_name="s",
                                    num_cores=2, num_subcores=16),
        compiler_params=pltpu.CompilerParams(needs_layout_passes=False,
                                             use_tc_tiling_on_sc=False),
        scratch_types=[
            pltpu.VMEM((pos_per_w,), jnp.int32),
            pltpu.VMEM((neg_per_w,), jnp.int32),
            pltpu.VMEM((pos_per_w,), jnp.float32),
            pltpu.VMEM((neg_per_w,), jnp.float32),
            pltpu.VMEM((2, LANES), jnp.float32),
            pltpu.SemaphoreType.DMA,
        ],
    )(scores, pos_v, neg_v)


def kernel(pos_u, pos_v, neg_v, batch_size, u_weight, v_weight):
    vocab = v_weight.shape[0]
    ut = u_weight.T
    vt = v_weight.T
    scores = _tc_scores(pos_u.astype(jnp.int32), ut, vt, vocab)
    parts = _sc_reduce(scores, pos_v, neg_v,
                       pos_per_w=pos_v.shape[0] // NW,
                       neg_per_w=neg_v.shape[0] // NW)
    pos_sum = jnp.sum(parts[:, 0, :])
    s = jnp.sum(parts[:, 1, :])
    return -pos_sum + jnp.log(1.0 + s)


# speedup vs baseline: 9.8976x; 1.7734x over previous
import functools

import jax
import jax.numpy as jnp
from jax import lax
from jax.experimental import pallas as pl
from jax.experimental.pallas import tpu as pltpu
from jax.experimental.pallas import tpu_sc as plsc

DIM = 64
LANES = 16
NW = 32
BLK = 16384
IDXW = 128


def _matvec_body(pu_ref, ut_blk, vt_blk, out_blk):
    j = pu_ref[0] % 128
    col = lax.broadcasted_iota(jnp.int32, (DIM, 128), 1)
    ucol = jnp.sum(jnp.where(col == j, ut_blk[...], 0.0), axis=1,
                   keepdims=True)
    out_blk[...] = jnp.sum(vt_blk[...] * ucol, axis=0)


def _tc_scores(pos_u, ut, vt, vocab):
    grid = (pl.cdiv(vocab, BLK),)
    return pl.pallas_call(
        _matvec_body,
        grid_spec=pltpu.PrefetchScalarGridSpec(
            num_scalar_prefetch=1,
            grid=grid,
            in_specs=[
                pl.BlockSpec((DIM, 128), lambda i, pu: (0, pu[0] // 128)),
                pl.BlockSpec((DIM, BLK), lambda i, pu: (0, i)),
            ],
            out_specs=pl.BlockSpec((BLK,), lambda i, pu: (i,)),
        ),
        out_shape=jax.ShapeDtypeStruct((vocab,), jnp.float32),
    )(pos_u, ut, vt)


def _sc_body(scores_hbm, posv_hbm, negv_hbm, out_hbm,
             pidx_v, nidx_v, pvals_v, nvals_v, part_v, sem,
             *, pos_per_w, neg_per_w):
    wid = lax.axis_index("s") * 2 + lax.axis_index("c")

    pltpu.sync_copy(posv_hbm.at[pl.ds(wid * pos_per_w, pos_per_w)], pidx_v)
    pltpu.sync_copy(negv_hbm.at[pl.ds(wid * neg_per_w, neg_per_w)], nidx_v)

    descs = []
    for j in range(pos_per_w // IDXW):
        descs.append(pltpu.async_copy(
            scores_hbm.at[pidx_v.at[pl.ds(j * IDXW, IDXW)]],
            pvals_v.at[pl.ds(j * IDXW, IDXW)], sem))
    for j in range(neg_per_w // IDXW):
        descs.append(pltpu.async_copy(
            scores_hbm.at[nidx_v.at[pl.ds(j * IDXW, IDXW)]],
            nvals_v.at[pl.ds(j * IDXW, IDXW)], sem))
    for d in descs:
        d.wait()

    zero = jnp.zeros((LANES,), jnp.float32)

    def pos_iter(i, acc):
        return acc + pvals_v[pl.ds(i * LANES, LANES)]

    pos_acc = lax.fori_loop(0, pos_per_w // LANES, pos_iter, zero)

    def neg_iter(i, acc):
        return acc + jnp.exp(nvals_v[pl.ds(i * LANES, LANES)])

    neg_acc = lax.fori_loop(0, neg_per_w // LANES, neg_iter, zero)

    part_v[0, :] = pos_acc
    part_v[1, :] = neg_acc
    pltpu.sync_copy(part_v, out_hbm.at[wid])


@functools.partial(jax.jit, static_argnames=("pos_per_w", "neg_per_w"))
def _sc_reduce(scores, pos_v, neg_v, *, pos_per_w, neg_per_w):
    body = functools.partial(_sc_body, pos_per_w=pos_per_w,
                             neg_per_w=neg_per_w)
    return pl.kernel(
        body,
        out_type=jax.ShapeDtypeStruct((NW, 2, LANES), jnp.float32),
        mesh=plsc.VectorSubcoreMesh(core_axis_name="c", subcore_axis_name="s",
                                    num_cores=2, num_subcores=16),
        compiler_params=pltpu.CompilerParams(needs_layout_passes=False,
                                             use_tc_tiling_on_sc=False),
        scratch_types=[
            pltpu.VMEM((pos_per_w,), jnp.int32),
            pltpu.VMEM((neg_per_w,), jnp.int32),
            pltpu.VMEM((pos_per_w,), jnp.float32),
            pltpu.VMEM((neg_per_w,), jnp.float32),
            pltpu.VMEM((2, LANES), jnp.float32),
            pltpu.SemaphoreType.DMA,
        ],
    )(scores, pos_v, neg_v)


def kernel(pos_u, pos_v, neg_v, batch_size, u_weight, v_weight):
    vocab = v_weight.shape[0]
    ut = u_weight.T
    vt = v_weight.T
    scores = _tc_scores(pos_u.astype(jnp.int32), ut, vt, vocab)
    parts = _sc_reduce(scores, pos_v, neg_v,
                       pos_per_w=pos_v.shape[0] // NW,
                       neg_per_w=neg_v.shape[0] // NW)
    pos_sum = jnp.sum(parts[:, 0, :])
    s = jnp.sum(parts[:, 1, :])
    return -pos_sum + jnp.log(1.0 + s)


# speedup vs baseline: 11.0529x; 1.1167x over previous
import functools

import jax
import jax.numpy as jnp
from jax import lax
from jax.experimental import pallas as pl
from jax.experimental.pallas import tpu as pltpu
from jax.experimental.pallas import tpu_sc as plsc

DIM = 64
LANES = 16
NW = 32
BLK = 65536
IDXW = 128


def _matvec_body(pu_ref, ut_blk, vt_blk, out_blk):
    j = pu_ref[0] % 128
    col = lax.broadcasted_iota(jnp.int32, (DIM, 128), 1)
    ucol = jnp.sum(jnp.where(col == j, ut_blk[...], 0.0), axis=1,
                   keepdims=True)
    out_blk[...] = jnp.sum(vt_blk[...] * ucol, axis=0)


def _tc_scores(pos_u, ut, vt, vocab):
    grid = (pl.cdiv(vocab, BLK),)
    return pl.pallas_call(
        _matvec_body,
        grid_spec=pltpu.PrefetchScalarGridSpec(
            num_scalar_prefetch=1,
            grid=grid,
            in_specs=[
                pl.BlockSpec((DIM, 128), lambda i, pu: (0, pu[0] // 128)),
                pl.BlockSpec((DIM, BLK), lambda i, pu: (0, i)),
            ],
            out_specs=pl.BlockSpec((BLK,), lambda i, pu: (i,)),
        ),
        out_shape=jax.ShapeDtypeStruct((vocab,), jnp.float32),
    )(pos_u, ut, vt)


def _sc_body(scores_hbm, posv_hbm, negv_hbm, out_hbm,
             pidx_v, nidx_v, pvals_v, nvals_v, part_v, sem,
             *, pos_per_w, neg_per_w):
    wid = lax.axis_index("s") * 2 + lax.axis_index("c")

    pltpu.sync_copy(posv_hbm.at[pl.ds(wid * pos_per_w, pos_per_w)], pidx_v)
    pltpu.sync_copy(negv_hbm.at[pl.ds(wid * neg_per_w, neg_per_w)], nidx_v)

    descs = []
    for j in range(pos_per_w // IDXW):
        descs.append(pltpu.async_copy(
            scores_hbm.at[pidx_v.at[pl.ds(j * IDXW, IDXW)]],
            pvals_v.at[pl.ds(j * IDXW, IDXW)], sem))
    for j in range(neg_per_w // IDXW):
        descs.append(pltpu.async_copy(
            scores_hbm.at[nidx_v.at[pl.ds(j * IDXW, IDXW)]],
            nvals_v.at[pl.ds(j * IDXW, IDXW)], sem))
    for d in descs:
        d.wait()

    zero = jnp.zeros((LANES,), jnp.float32)

    def pos_iter(i, acc):
        return acc + pvals_v[pl.ds(i * LANES, LANES)]

    pos_acc = lax.fori_loop(0, pos_per_w // LANES, pos_iter, zero)

    def neg_iter(i, acc):
        return acc + jnp.exp(nvals_v[pl.ds(i * LANES, LANES)])

    neg_acc = lax.fori_loop(0, neg_per_w // LANES, neg_iter, zero)

    part_v[0, :] = pos_acc
    part_v[1, :] = neg_acc
    pltpu.sync_copy(part_v, out_hbm.at[wid])


@functools.partial(jax.jit, static_argnames=("pos_per_w", "neg_per_w"))
def _sc_reduce(scores, pos_v, neg_v, *, pos_per_w, neg_per_w):
    body = functools.partial(_sc_body, pos_per_w=pos_per_w,
                             neg_per_w=neg_per_w)
    return pl.kernel(
        body,
        out_type=jax.ShapeDtypeStruct((NW, 2, LANES), jnp.float32),
        mesh=plsc.VectorSubcoreMesh(core_axis_name="c", subcore_axis_name="s",
                                    num_cores=2, num_subcores=16),
        compiler_params=pltpu.CompilerParams(needs_layout_passes=False,
                                             use_tc_tiling_on_sc=False),
        scratch_types=[
            pltpu.VMEM((pos_per_w,), jnp.int32),
            pltpu.VMEM((neg_per_w,), jnp.int32),
            pltpu.VMEM((pos_per_w,), jnp.float32),
            pltpu.VMEM((neg_per_w,), jnp.float32),
            pltpu.VMEM((2, LANES), jnp.float32),
            pltpu.SemaphoreType.DMA,
        ],
    )(scores, pos_v, neg_v)


def kernel(pos_u, pos_v, neg_v, batch_size, u_weight, v_weight):
    vocab = v_weight.shape[0]
    ut = u_weight.T
    vt = v_weight.T
    scores = _tc_scores(pos_u.astype(jnp.int32), ut, vt, vocab)
    parts = _sc_reduce(scores, pos_v, neg_v,
                       pos_per_w=pos_v.shape[0] // NW,
                       neg_per_w=neg_v.shape[0] // NW)
    pos_sum = jnp.sum(parts[:, 0, :])
    s = jnp.sum(parts[:, 1, :])
    return -pos_sum + jnp.log(1.0 + s)


# speedup vs baseline: 11.1683x; 1.0104x over previous
import functools

import jax
import jax.numpy as jnp
from jax import lax
from jax.experimental import pallas as pl
from jax.experimental.pallas import tpu as pltpu
from jax.experimental.pallas import tpu_sc as plsc

DIM = 64
LANES = 16
NW = 32
BLK = 98304
IDXW = 128


def _matvec_body(pu_ref, ut_blk, vt_blk, out_blk):
    j = pu_ref[0] % 128
    col = lax.broadcasted_iota(jnp.int32, (DIM, 128), 1)
    ucol = jnp.sum(jnp.where(col == j, ut_blk[...], 0.0), axis=1,
                   keepdims=True)
    out_blk[...] = lax.dot_general(
        ucol, vt_blk[...], (((0,), (0,)), ((), ())),
        preferred_element_type=jnp.float32)[0, :]


def _tc_scores(pos_u, ut, vt, vocab):
    grid = (pl.cdiv(vocab, BLK),)
    return pl.pallas_call(
        _matvec_body,
        grid_spec=pltpu.PrefetchScalarGridSpec(
            num_scalar_prefetch=1,
            grid=grid,
            in_specs=[
                pl.BlockSpec((DIM, 128), lambda i, pu: (0, pu[0] // 128)),
                pl.BlockSpec((DIM, BLK), lambda i, pu: (0, i)),
            ],
            out_specs=pl.BlockSpec((BLK,), lambda i, pu: (i,)),
        ),
        out_shape=jax.ShapeDtypeStruct((vocab,), jnp.float32),
        compiler_params=pltpu.CompilerParams(vmem_limit_bytes=63 * 1024 * 1024),
    )(pos_u, ut, vt)


def _sc_body(scores_hbm, posv_hbm, negv_hbm, out_hbm,
             pidx_v, nidx_v, pvals_v, nvals_v, part_v, sem,
             *, pos_per_w, neg_per_w):
    wid = lax.axis_index("s") * 2 + lax.axis_index("c")

    pltpu.sync_copy(posv_hbm.at[pl.ds(wid * pos_per_w, pos_per_w)], pidx_v)
    pltpu.sync_copy(negv_hbm.at[pl.ds(wid * neg_per_w, neg_per_w)], nidx_v)

    descs = []
    for j in range(pos_per_w // IDXW):
        descs.append(pltpu.async_copy(
            scores_hbm.at[pidx_v.at[pl.ds(j * IDXW, IDXW)]],
            pvals_v.at[pl.ds(j * IDXW, IDXW)], sem))
    for j in range(neg_per_w // IDXW):
        descs.append(pltpu.async_copy(
            scores_hbm.at[nidx_v.at[pl.ds(j * IDXW, IDXW)]],
            nvals_v.at[pl.ds(j * IDXW, IDXW)], sem))
    for d in descs:
        d.wait()

    zero = jnp.zeros((LANES,), jnp.float32)

    def pos_iter(i, acc):
        return acc + pvals_v[pl.ds(i * LANES, LANES)]

    pos_acc = lax.fori_loop(0, pos_per_w // LANES, pos_iter, zero)

    def neg_iter(i, acc):
        return acc + jnp.exp(nvals_v[pl.ds(i * LANES, LANES)])

    neg_acc = lax.fori_loop(0, neg_per_w // LANES, neg_iter, zero)

    part_v[0, :] = pos_acc
    part_v[1, :] = neg_acc
    pltpu.sync_copy(part_v, out_hbm.at[wid])


@functools.partial(jax.jit, static_argnames=("pos_per_w", "neg_per_w"))
def _sc_reduce(scores, pos_v, neg_v, *, pos_per_w, neg_per_w):
    body = functools.partial(_sc_body, pos_per_w=pos_per_w,
                             neg_per_w=neg_per_w)
    return pl.kernel(
        body,
        out_type=jax.ShapeDtypeStruct((NW, 2, LANES), jnp.float32),
        mesh=plsc.VectorSubcoreMesh(core_axis_name="c", subcore_axis_name="s",
                                    num_cores=2, num_subcores=16),
        compiler_params=pltpu.CompilerParams(needs_layout_passes=False,
                                             use_tc_tiling_on_sc=False),
        scratch_types=[
            pltpu.VMEM((pos_per_w,), jnp.int32),
            pltpu.VMEM((neg_per_w,), jnp.int32),
            pltpu.VMEM((pos_per_w,), jnp.float32),
            pltpu.VMEM((neg_per_w,), jnp.float32),
            pltpu.VMEM((2, LANES), jnp.float32),
            pltpu.SemaphoreType.DMA,
        ],
    )(scores, pos_v, neg_v)


def kernel(pos_u, pos_v, neg_v, batch_size, u_weight, v_weight):
    vocab = v_weight.shape[0]
    ut = u_weight.T
    vt = v_weight.T
    scores = _tc_scores(pos_u.astype(jnp.int32), ut, vt, vocab)
    parts = _sc_reduce(scores, pos_v, neg_v,
                       pos_per_w=pos_v.shape[0] // NW,
                       neg_per_w=neg_v.shape[0] // NW)
    pos_sum = jnp.sum(parts[:, 0, :])
    s = jnp.sum(parts[:, 1, :])
    return -pos_sum + jnp.log(1.0 + s)
